# speedup vs baseline: 12.1211x; 12.1211x over previous
import jax
import jax.numpy as jnp
from jax.experimental import pallas as pl
from jax.experimental.pallas import tpu as pltpu

_B = 128
_N = 32768
_R = 8


def _body(k_ref, x_ref, o_ref):
    k = k_ref[0]
    x = x_ref[...]
    bits = jax.lax.bitcast_convert_type(x, jnp.int32) & jnp.int32(0x7FFFFFFF)

    def step(i, t):
        cand = t | (jnp.int32(1) << (jnp.int32(30) - i))
        cnt = jnp.sum((bits >= cand).astype(jnp.int32), axis=1, keepdims=True)
        return jnp.where(cnt >= k, cand, t)

    t = jax.lax.fori_loop(0, 31, step, jnp.zeros((_R, 1), jnp.int32))
    o_ref[...] = jnp.where(bits >= t, x, jnp.float32(0.0))


def kernel(scores, k):
    kk = jnp.asarray(k, jnp.int32).reshape(1)
    return pl.pallas_call(
        _body,
        grid=(_B // _R,),
        in_specs=[
            pl.BlockSpec(memory_space=pltpu.SMEM),
            pl.BlockSpec((_R, _N), lambda i: (i, 0)),
        ],
        out_specs=pl.BlockSpec((_R, _N), lambda i: (i, 0)),
        out_shape=jax.ShapeDtypeStruct((_B, _N), jnp.float32),
    )(kk, scores)
